# Initial kernel scaffold; baseline (speedup 1.0000x reference)
import jax
import jax.numpy as jnp
from jax.experimental import pallas as pl


def kernel(W1, b1, W2, b2, temb, cls_emb, noise, labels):
    raise NotImplementedError("write your pallas kernel here")


# speedup vs baseline: 1.2968x; 1.2968x over previous
import functools

import numpy as np
import jax
import jax.numpy as jnp
from jax.experimental import pallas as pl
from jax.experimental.pallas import tpu as pltpu

_C = 3
_HID = 32
_T = 8
_NB = 4

_betas = np.linspace(1e-4, 2e-2, _T, dtype=np.float64)
_alphas = 1.0 - _betas
_abar = np.cumprod(_alphas)
_abar_prev = np.append(1.0, _abar[:-1])
_order = np.arange(_T - 1, -1, -1)

_C0 = np.sqrt(1.0 / _abar)[_order]
_C1 = np.sqrt(1.0 / _abar - 1.0)[_order]
_CA = (_betas * np.sqrt(_abar_prev) / (1.0 - _abar))[_order]
_CB = ((1.0 - _abar_prev) * np.sqrt(_alphas) / (1.0 - _abar))[_order]
_CD = np.sqrt(_betas * (1.0 - _abar_prev) / (1.0 - _abar))
_CD[0] = 0.0
_CD = _CD[_order]
_CD_SQRT2 = _CD * np.sqrt(2.0)

_N_NOISE = int(np.sum(_CD != 0.0))

_U_LO = float(np.nextafter(np.float32(-1.0), np.float32(0.0)))
_U_SCALE = float(np.float32(1.0) - np.float32(_U_LO))

_KS2 = np.uint32(0x1BD11BDA ^ 0 ^ 1)


def _rotl(v, r):
    return (v << np.uint32(r)) | (v >> np.uint32(32 - r))


def _threefry_bits(cnt):
    x1 = cnt + np.uint32(1)
    x0 = x1
    x1 = _rotl(x1, 13) ^ x0
    for r in (15, 26, 6):
        x0 = x0 + x1
        x1 = _rotl(x1, r) ^ x0
    x0 = x0 + np.uint32(1)
    x1 = x1 + (_KS2 + np.uint32(1))
    for r in (17, 29, 16, 24):
        x0 = x0 + x1
        x1 = _rotl(x1, r) ^ x0
    x0 = x0 + _KS2
    x1 = x1 + np.uint32(2)
    for r in (13, 15, 26, 6):
        x0 = x0 + x1
        x1 = _rotl(x1, r) ^ x0
    x1 = x1 + np.uint32(4)
    for r in (17, 29, 16, 24):
        x0 = x0 + x1
        x1 = _rotl(x1, r) ^ x0
    x0 = x0 + np.uint32(1)
    x1 = x1 + (_KS2 + np.uint32(4))
    for r in (13, 15, 26, 6):
        x0 = x0 + x1
        x1 = _rotl(x1, r) ^ x0
    x0 = x0 + _KS2
    x1 = x1 + np.uint32(5)
    return x0 ^ x1


def _erfinv_poly(u):
    w = -jnp.log1p(-u * u)
    wl = w - np.float32(2.5)
    p1 = jnp.float32(2.81022636e-08)
    for c in (3.43273939e-07, -3.5233877e-06, -4.39150654e-06, 0.00021858087,
              -0.00125372503, -0.00417768164, 0.246640727, 1.50140941):
        p1 = p1 * wl + np.float32(c)
    wg = jnp.sqrt(w) - np.float32(3.0)
    p2 = jnp.float32(-0.000200214257)
    for c in (0.000100950558, 0.00134934322, -0.00367342844, 0.00573950773,
              -0.0076224613, 0.00943887047, 1.00167406, 2.83297682):
        p2 = p2 * wg + np.float32(c)
    return jnp.where(w < np.float32(5.0), p1, p2) * u


def _sample_body(x_ref, w1_ref, w2_ref, out_ref, *, nb, s_dim, l_dim):
    hw = s_dim * l_dim
    g = pl.program_id(0)

    shape = (nb, _N_NOISE, _C, s_dim, l_dim)
    cnt = (jax.lax.broadcasted_iota(jnp.int32, shape, 0) * (_T * _C * hw)
           + jax.lax.broadcasted_iota(jnp.int32, shape, 1) * (_C * hw)
           + jax.lax.broadcasted_iota(jnp.int32, shape, 2) * hw
           + jax.lax.broadcasted_iota(jnp.int32, shape, 3) * l_dim
           + jax.lax.broadcasted_iota(jnp.int32, shape, 4)
           + g * (nb * _T * _C * hw))
    bits = _threefry_bits(pltpu.bitcast(cnt, jnp.uint32))
    fbits = (bits >> np.uint32(9)) | np.uint32(0x3F800000)
    frac = pltpu.bitcast(fbits, jnp.float32) - np.float32(1.0)
    u = jnp.maximum(np.float32(_U_LO),
                    frac * np.float32(_U_SCALE) + np.float32(_U_LO))
    z_all = _erfinv_poly(u)

    ones = jnp.ones((1, s_dim, l_dim), jnp.float32)
    for i in range(nb):
        x = x_ref[i]
        for s in range(_T):
            xa = jnp.concatenate([x, ones], axis=0)
            hpre = jnp.einsum("hc,csl->hsl", w1_ref[i, s], xa,
                              preferred_element_type=jnp.float32)
            h = hpre * jax.nn.sigmoid(hpre)
            ha = jnp.concatenate([h, ones], axis=0)
            psi = jnp.einsum("ch,hsl->csl", w2_ref[s], ha,
                             preferred_element_type=jnp.float32)
            pred = jax.lax.clamp(jnp.float32(-1.0),
                                 np.float32(_C0[s]) * x + psi,
                                 jnp.float32(1.0))
            xn = np.float32(_CA[s]) * pred
            if _CB[s] != 0.0:
                xn = xn + np.float32(_CB[s]) * x
            if _CD[s] != 0.0:
                xn = xn + np.float32(_CD_SQRT2[s]) * z_all[i, s]
            x = xn
        out_ref[i] = x


def kernel(W1, b1, W2, b2, temb, cls_emb, noise, labels):
    B = noise.shape[0]
    H, W = noise.shape[2], noise.shape[3]
    hw = H * W
    s_dim = 8
    l_dim = hw // s_dim

    x_init = noise.astype(jnp.float32).reshape(B, _C, s_dim, l_dim)

    cls = cls_emb[labels.astype(jnp.int32)]
    temb_o = temb[::-1]
    bias = (b1[None, None, :] + temb_o[None, :, :]
            + cls[:, None, :])
    w1t = jnp.transpose(W1)
    w1aug = jnp.concatenate(
        [jnp.broadcast_to(w1t[None, None], (B, _T, _HID, _C)),
         bias[..., None]], axis=3)

    w2t = jnp.transpose(W2)
    scale = jnp.asarray(-_C1, jnp.float32)
    w2aug = jnp.concatenate([w2t[None], b2[None, :, None]],
                            axis=2)
    w2tab = scale[:, None, None] * w2aug

    body = functools.partial(_sample_body, nb=_NB, s_dim=s_dim, l_dim=l_dim)
    n_px = B * hw
    grid = (B // _NB,)
    out = pl.pallas_call(
        body,
        grid=grid,
        in_specs=[
            pl.BlockSpec((_NB, _C, s_dim, l_dim), lambda gi: (gi, 0, 0, 0)),
            pl.BlockSpec((_NB, _T, _HID, _C + 1), lambda gi: (gi, 0, 0, 0)),
            pl.BlockSpec((_T, _C, _HID + 1), lambda gi: (0, 0, 0)),
        ],
        out_specs=pl.BlockSpec((_NB, _C, s_dim, l_dim), lambda gi: (gi, 0, 0, 0)),
        out_shape=jax.ShapeDtypeStruct((B, _C, s_dim, l_dim), jnp.float32),
        compiler_params=pltpu.CompilerParams(
            dimension_semantics=("parallel",)),
        cost_estimate=pl.CostEstimate(
            flops=int(_T * n_px * (2 * (_C + 1) * _HID + 2 * (_HID + 1) * _C
                                   + 16 * _C) + _N_NOISE * n_px * _C * 40),
            transcendentals=int(_T * n_px * _HID + 2 * _N_NOISE * n_px * _C),
            bytes_accessed=int(4 * (2 * B * _C * hw + B * _T * _HID * (_C + 1)
                                    + _T * _C * (_HID + 1))),
        ),
    )(x_init, w1aug, w2tab)
    return out.reshape(B, _C, H, W)


# speedup vs baseline: 1.5722x; 1.2124x over previous
import functools

import numpy as np
import jax
import jax.numpy as jnp
from jax.experimental import pallas as pl
from jax.experimental.pallas import tpu as pltpu

_C = 3
_HID = 32
_T = 8
_LC = 512

_betas = np.linspace(1e-4, 2e-2, _T, dtype=np.float64)
_alphas = 1.0 - _betas
_abar = np.cumprod(_alphas)
_abar_prev = np.append(1.0, _abar[:-1])
_order = np.arange(_T - 1, -1, -1)

_C0 = np.sqrt(1.0 / _abar)[_order]
_C1 = np.sqrt(1.0 / _abar - 1.0)[_order]
_CA = (_betas * np.sqrt(_abar_prev) / (1.0 - _abar))[_order]
_CB = ((1.0 - _abar_prev) * np.sqrt(_alphas) / (1.0 - _abar))[_order]
_CD = np.sqrt(_betas * (1.0 - _abar_prev) / (1.0 - _abar))
_CD[0] = 0.0
_CD = _CD[_order]
_CD_SQRT2 = _CD * np.sqrt(2.0)

_U_LO = float(np.nextafter(np.float32(-1.0), np.float32(0.0)))
_U_SCALE = float(np.float32(1.0) - np.float32(_U_LO))

_KS2 = np.uint32(0x1BD11BDA ^ 0 ^ 1)


def _rotl(v, r):
    return (v << np.uint32(r)) | (v >> np.uint32(32 - r))


def _threefry_bits(cnt):
    x1 = cnt + np.uint32(1)
    x0 = x1
    x1 = _rotl(x1, 13) ^ x0
    for r in (15, 26, 6):
        x0 = x0 + x1
        x1 = _rotl(x1, r) ^ x0
    x0 = x0 + np.uint32(1)
    x1 = x1 + (_KS2 + np.uint32(1))
    for r in (17, 29, 16, 24):
        x0 = x0 + x1
        x1 = _rotl(x1, r) ^ x0
    x0 = x0 + _KS2
    x1 = x1 + np.uint32(2)
    for r in (13, 15, 26, 6):
        x0 = x0 + x1
        x1 = _rotl(x1, r) ^ x0
    x1 = x1 + np.uint32(4)
    for r in (17, 29, 16, 24):
        x0 = x0 + x1
        x1 = _rotl(x1, r) ^ x0
    x0 = x0 + np.uint32(1)
    x1 = x1 + (_KS2 + np.uint32(4))
    for r in (13, 15, 26, 6):
        x0 = x0 + x1
        x1 = _rotl(x1, r) ^ x0
    x0 = x0 + _KS2
    x1 = x1 + np.uint32(5)
    return x0 ^ x1


def _erfinv_poly(u):
    w = -jnp.log1p(-u * u)
    wl = w - np.float32(2.5)
    p1 = jnp.float32(2.81022636e-08)
    for c in (3.43273939e-07, -3.5233877e-06, -4.39150654e-06, 0.00021858087,
              -0.00125372503, -0.00417768164, 0.246640727, 1.50140941):
        p1 = p1 * wl + np.float32(c)
    wg = jnp.sqrt(w) - np.float32(3.0)
    p2 = jnp.float32(-0.000200214257)
    for c in (0.000100950558, 0.00134934322, -0.00367342844, 0.00573950773,
              -0.0076224613, 0.00943887047, 1.00167406, 2.83297682):
        p2 = p2 * wg + np.float32(c)
    return jnp.where(w < np.float32(5.0), p1, p2) * u


def _gen_z(base, ch, s_dim, l_dim):
    shape = (_C, s_dim, _LC)
    cnt = (jax.lax.broadcasted_iota(jnp.int32, shape, 0) * (s_dim * l_dim)
           + jax.lax.broadcasted_iota(jnp.int32, shape, 1) * l_dim
           + jax.lax.broadcasted_iota(jnp.int32, shape, 2)
           + (base + ch * _LC))
    bits = _threefry_bits(pltpu.bitcast(cnt, jnp.uint32))
    fbits = (bits >> np.uint32(9)) | np.uint32(0x3F800000)
    frac = pltpu.bitcast(fbits, jnp.float32) - np.float32(1.0)
    u = jnp.maximum(np.float32(_U_LO),
                    frac * np.float32(_U_SCALE) + np.float32(_U_LO))
    return _erfinv_poly(u)


def _sample_body(x_ref, w1_ref, w2_ref, out_ref, *, s_dim, l_dim):
    hw = s_dim * l_dim
    nch = l_dim // _LC
    g = pl.program_id(0)
    ones = jnp.ones((1, s_dim, _LC), jnp.float32)

    xs = [x_ref[0, :, :, ch * _LC:(ch + 1) * _LC] for ch in range(nch)]
    for s in range(_T):
        w1s = w1_ref[0, s]
        w2s = w2_ref[s]
        zbase = (g * _T + s) * (_C * hw)
        for ch in range(nch):
            x = xs[ch]
            xa = jnp.concatenate([x, ones], axis=0)
            hpre = jnp.einsum("hc,csl->hsl", w1s, xa,
                              preferred_element_type=jnp.float32)
            h = hpre * jax.nn.sigmoid(hpre)
            ha = jnp.concatenate([h, ones], axis=0)
            psi = jnp.einsum("ch,hsl->csl", w2s, ha,
                             preferred_element_type=jnp.float32)
            pred = jax.lax.clamp(jnp.float32(-1.0),
                                 np.float32(_C0[s]) * x + psi,
                                 jnp.float32(1.0))
            xn = np.float32(_CA[s]) * pred
            if _CB[s] != 0.0:
                xn = xn + np.float32(_CB[s]) * x
            if _CD[s] != 0.0:
                z = _gen_z(zbase, ch, s_dim, l_dim)
                xn = xn + np.float32(_CD_SQRT2[s]) * z
            xs[ch] = xn
    for ch in range(nch):
        out_ref[0, :, :, ch * _LC:(ch + 1) * _LC] = xs[ch]


def kernel(W1, b1, W2, b2, temb, cls_emb, noise, labels):
    B = noise.shape[0]
    H, W = noise.shape[2], noise.shape[3]
    hw = H * W
    s_dim = 8
    l_dim = hw // s_dim

    x_init = noise.astype(jnp.float32).reshape(B, _C, s_dim, l_dim)

    cls = cls_emb[labels.astype(jnp.int32)]
    temb_o = temb[::-1]
    bias = (b1[None, None, :] + temb_o[None, :, :]
            + cls[:, None, :])
    w1t = jnp.transpose(W1)
    w1aug = jnp.concatenate(
        [jnp.broadcast_to(w1t[None, None], (B, _T, _HID, _C)),
         bias[..., None]], axis=3)

    w2t = jnp.transpose(W2)
    scale = jnp.asarray(-_C1, jnp.float32)
    w2aug = jnp.concatenate([w2t[None], b2[None, :, None]],
                            axis=2)
    w2tab = scale[:, None, None] * w2aug

    body = functools.partial(_sample_body, s_dim=s_dim, l_dim=l_dim)
    n_px = B * hw
    n_noise = int(np.sum(_CD != 0.0))
    out = pl.pallas_call(
        body,
        grid=(B,),
        in_specs=[
            pl.BlockSpec((1, _C, s_dim, l_dim), lambda gi: (gi, 0, 0, 0)),
            pl.BlockSpec((1, _T, _HID, _C + 1), lambda gi: (gi, 0, 0, 0)),
            pl.BlockSpec((_T, _C, _HID + 1), lambda gi: (0, 0, 0)),
        ],
        out_specs=pl.BlockSpec((1, _C, s_dim, l_dim), lambda gi: (gi, 0, 0, 0)),
        out_shape=jax.ShapeDtypeStruct((B, _C, s_dim, l_dim), jnp.float32),
        compiler_params=pltpu.CompilerParams(
            dimension_semantics=("parallel",)),
        cost_estimate=pl.CostEstimate(
            flops=int(_T * n_px * (2 * (_C + 1) * _HID + 2 * (_HID + 1) * _C
                                   + 16 * _C) + n_noise * n_px * _C * 40),
            transcendentals=int(_T * n_px * _HID + 2 * n_noise * n_px * _C),
            bytes_accessed=int(4 * (2 * B * _C * hw + B * _T * _HID * (_C + 1)
                                    + _T * _C * (_HID + 1))),
        ),
    )(x_init, w1aug, w2tab)
    return out.reshape(B, _C, H, W)


# speedup vs baseline: 2.0002x; 1.2722x over previous
import functools

import numpy as np
import jax
import jax.numpy as jnp
from jax.experimental import pallas as pl
from jax.experimental.pallas import tpu as pltpu

_C = 3
_HID = 32
_T = 8
_LC = 2048

_betas = np.linspace(1e-4, 2e-2, _T, dtype=np.float64)
_alphas = 1.0 - _betas
_abar = np.cumprod(_alphas)
_abar_prev = np.append(1.0, _abar[:-1])
_order = np.arange(_T - 1, -1, -1)

_C0 = np.sqrt(1.0 / _abar)[_order]
_C1 = np.sqrt(1.0 / _abar - 1.0)[_order]
_CA = (_betas * np.sqrt(_abar_prev) / (1.0 - _abar))[_order]
_CB = ((1.0 - _abar_prev) * np.sqrt(_alphas) / (1.0 - _abar))[_order]
_CD = np.sqrt(_betas * (1.0 - _abar_prev) / (1.0 - _abar))
_CD[0] = 0.0
_CD = _CD[_order]
_CD_SQRT2 = _CD * np.sqrt(2.0)

_CA_ROWS = [np.array([[v]] * 6 + [[1.0]] + [[0.0]], np.float32) for v in _CA]
_CB_ROWS = [np.array([[v]] * 6 + [[0.0]] + [[0.0]], np.float32) for v in _CB]
_CD_ROWS = [np.array([[v]] * 6 + [[0.0]] + [[0.0]], np.float32)
            for v in _CD_SQRT2]

_ALPHA = 1.0
for _ in range(80):
    _s = 1.0 / (1.0 + np.exp(-_ALPHA))
    _f = _ALPHA * _s - 1.0
    _fp = _s + _ALPHA * _s * (1.0 - _s)
    _ALPHA = _ALPHA - _f / _fp
_ALPHA = float(_ALPHA)

_U_LO = float(np.nextafter(np.float32(-1.0), np.float32(0.0)))
_U_SCALE = float(np.float32(1.0) - np.float32(_U_LO))

_KS2 = np.uint32(0x1BD11BDA ^ 0 ^ 1)


def _rotl(v, r):
    return (v << np.uint32(r)) | (v >> np.uint32(32 - r))


def _threefry_bits(cnt):
    x1 = cnt + np.uint32(1)
    x0 = x1
    x1 = _rotl(x1, 13) ^ x0
    for r in (15, 26, 6):
        x0 = x0 + x1
        x1 = _rotl(x1, r) ^ x0
    x0 = x0 + np.uint32(1)
    x1 = x1 + (_KS2 + np.uint32(1))
    for r in (17, 29, 16, 24):
        x0 = x0 + x1
        x1 = _rotl(x1, r) ^ x0
    x0 = x0 + _KS2
    x1 = x1 + np.uint32(2)
    for r in (13, 15, 26, 6):
        x0 = x0 + x1
        x1 = _rotl(x1, r) ^ x0
    x1 = x1 + np.uint32(4)
    for r in (17, 29, 16, 24):
        x0 = x0 + x1
        x1 = _rotl(x1, r) ^ x0
    x0 = x0 + np.uint32(1)
    x1 = x1 + (_KS2 + np.uint32(4))
    for r in (13, 15, 26, 6):
        x0 = x0 + x1
        x1 = _rotl(x1, r) ^ x0
    x0 = x0 + _KS2
    x1 = x1 + np.uint32(5)
    return x0 ^ x1


def _erfinv_poly(u):
    w = -jnp.log1p(-u * u)
    wl = w - np.float32(2.5)
    p1 = jnp.float32(2.81022636e-08)
    for c in (3.43273939e-07, -3.5233877e-06, -4.39150654e-06, 0.00021858087,
              -0.00125372503, -0.00417768164, 0.246640727, 1.50140941):
        p1 = p1 * wl + np.float32(c)
    wg = jnp.sqrt(w) - np.float32(3.0)
    p2 = jnp.float32(-0.000200214257)
    for c in (0.000100950558, 0.00134934322, -0.00367342844, 0.00573950773,
              -0.0076224613, 0.00943887047, 1.00167406, 2.83297682):
        p2 = p2 * wg + np.float32(c)
    return jnp.where(w < np.float32(5.0), p1, p2) * u


def _gen_z(rowoff, base, nsteps):
    shape = (nsteps, 8, _LC)
    cnt = (rowoff
           + jax.lax.broadcasted_iota(jnp.int32, shape, 2)
           + base)
    bits = _threefry_bits(pltpu.bitcast(cnt, jnp.uint32))
    fbits = (bits >> np.uint32(9)) | np.uint32(0x3F800000)
    frac = pltpu.bitcast(fbits, jnp.float32) - np.float32(1.0)
    u = frac * np.float32(_U_SCALE) + np.float32(_U_LO)
    return _erfinv_poly(u)


def _sample_body(x_ref, w1_ref, w2_ref, coef_ref, roff_ref, out_ref, *, hw):
    nch = hw // _LC
    g = pl.program_id(0)
    gbase = g * np.int32(2 * _T * _C * hw)

    xs = [x_ref[0, :, ch * _LC:(ch + 1) * _LC] for ch in range(nch)]
    zhold = [None] * nch
    for s in range(_T):
        w1s = w1_ref[0, s]
        w2s = w2_ref[s]
        cav = coef_ref[s, 0]
        cbv = coef_ref[s, 1]
        cdv = coef_ref[s, 2]
        for ch in range(nch):
            x = xs[ch]
            hpre = jnp.einsum("hk,kl->hl", w1s, x,
                              preferred_element_type=jnp.float32)
            sg = np.float32(0.5) * jnp.tanh(np.float32(0.5) * hpre) \
                + np.float32(0.5)
            h = hpre * sg
            psi = jnp.einsum("ch,hl->cl", w2s, h,
                             preferred_element_type=jnp.float32)
            pred = jax.lax.clamp(jnp.float32(-1.0),
                                 np.float32(_C0[s]) * x + psi,
                                 jnp.float32(1.0))
            xn = cav * pred
            if s < _T - 1:
                if s % 2 == 0:
                    nz = 2 if s < 6 else 1
                    zz = _gen_z(roff_ref[s // 2, 0:nz],
                                gbase + ch * _LC, nz)
                    z = zz[0]
                    if nz == 2:
                        zhold[ch] = zz[1]
                else:
                    z = zhold[ch]
                xn = xn + cbv * x + cdv * z
            xs[ch] = xn
    for ch in range(nch):
        out_ref[0, :, ch * _LC:(ch + 1) * _LC] = xs[ch]


def kernel(W1, b1, W2, b2, temb, cls_emb, noise, labels):
    B = noise.shape[0]
    H, W = noise.shape[2], noise.shape[3]
    hw = H * W
    B2 = B // 2

    noise6 = noise.astype(jnp.float32).reshape(B2, 6, hw)
    x2 = jnp.concatenate(
        [noise6, jnp.ones((B2, 1, hw), jnp.float32),
         jnp.zeros((B2, 1, hw), jnp.float32)], axis=1)

    cls = cls_emb[labels.astype(jnp.int32)]
    temb_o = temb[::-1]
    bias = (b1[None, None, :] + temb_o[None, :, :]
            + cls[:, None, :])
    biasA = bias[0::2]
    biasB = bias[1::2]
    w1t = jnp.transpose(W1)
    w1b = jnp.broadcast_to(w1t[None, None], (B2, _T, _HID, _C))
    zc3 = jnp.zeros((B2, _T, _HID, _C), jnp.float32)
    zc1 = jnp.zeros((B2, _T, _HID, 1), jnp.float32)
    rowsA = jnp.concatenate([w1b, zc3, biasA[..., None], zc1], axis=3)
    rowsB = jnp.concatenate([zc3, w1b, biasB[..., None], zc1], axis=3)
    alpha_row = jnp.broadcast_to(
        jnp.asarray([0, 0, 0, 0, 0, 0, _ALPHA, 0], jnp.float32)[None, None,
                                                                None, :],
        (B2, _T, 1, 8))
    w1tab = jnp.concatenate([rowsA, rowsB, alpha_row], axis=2)

    w2t = jnp.transpose(W2)
    z32 = jnp.zeros((_C, _HID), jnp.float32)
    b2c = b2[:, None]
    rA = jnp.concatenate([w2t, z32, b2c], axis=1)
    rB = jnp.concatenate([z32, w2t, b2c], axis=1)
    w2base = jnp.concatenate([rA, rB, jnp.zeros((2, 65), jnp.float32)],
                             axis=0)
    w2tab = (jnp.asarray(-_C1, jnp.float32)[:, None, None]
             * w2base[None])

    coef_tab = jnp.asarray(
        np.stack([np.stack([_CA_ROWS[s], _CB_ROWS[s], _CD_ROWS[s]], axis=0)
                  for s in range(_T)], axis=0))

    D = _T * _C * hw
    chan = np.array([0, hw, 2 * hw, D, D + hw, D + 2 * hw, 0, 0], np.int64)
    roff_tab = jnp.asarray(np.stack(
        [np.stack([chan + (s + p) * _C * hw for p in range(2)], axis=0)
         for s in (0, 2, 4, 6)], axis=0)[:, :, :, None].astype(np.int32))

    body = functools.partial(_sample_body, hw=hw)
    n_px = B * hw
    n_noise = int(np.sum(_CD != 0.0))
    out = pl.pallas_call(
        body,
        grid=(B2,),
        in_specs=[
            pl.BlockSpec((1, 8, hw), lambda gi: (gi, 0, 0)),
            pl.BlockSpec((1, _T, 2 * _HID + 1, 8), lambda gi: (gi, 0, 0, 0)),
            pl.BlockSpec((_T, 8, 2 * _HID + 1), lambda gi: (0, 0, 0)),
            pl.BlockSpec((_T, 3, 8, 1), lambda gi: (0, 0, 0, 0)),
            pl.BlockSpec((4, 2, 8, 1), lambda gi: (0, 0, 0, 0)),
        ],
        out_specs=pl.BlockSpec((1, 8, hw), lambda gi: (gi, 0, 0)),
        out_shape=jax.ShapeDtypeStruct((B2, 8, hw), jnp.float32),
        compiler_params=pltpu.CompilerParams(
            dimension_semantics=("parallel",)),
        cost_estimate=pl.CostEstimate(
            flops=int(_T * n_px * (2 * 8 * _HID + 2 * (_HID + 1) * _C
                                   + 16 * _C) + n_noise * n_px * _C * 45),
            transcendentals=int(_T * n_px * _HID + 2 * n_noise * n_px * _C),
            bytes_accessed=int(4 * (2 * B2 * 8 * hw
                                    + B2 * _T * (2 * _HID + 1) * 8
                                    + _T * 8 * (2 * _HID + 1))),
        ),
    )(x2, w1tab, w2tab, coef_tab, roff_tab)
    return out[:, 0:6, :].reshape(B, _C, H, W)


# speedup vs baseline: 2.0081x; 1.0040x over previous
import functools

import numpy as np
import jax
import jax.numpy as jnp
from jax.experimental import pallas as pl
from jax.experimental.pallas import tpu as pltpu

_C = 3
_HID = 32
_T = 8
_LC = 2048

_betas = np.linspace(1e-4, 2e-2, _T, dtype=np.float64)
_alphas = 1.0 - _betas
_abar = np.cumprod(_alphas)
_abar_prev = np.append(1.0, _abar[:-1])
_order = np.arange(_T - 1, -1, -1)

_C0 = np.sqrt(1.0 / _abar)[_order]
_C1 = np.sqrt(1.0 / _abar - 1.0)[_order]
_CA = (_betas * np.sqrt(_abar_prev) / (1.0 - _abar))[_order]
_CB = ((1.0 - _abar_prev) * np.sqrt(_alphas) / (1.0 - _abar))[_order]
_CD = np.sqrt(_betas * (1.0 - _abar_prev) / (1.0 - _abar))
_CD[0] = 0.0
_CD = _CD[_order]
_CD_SQRT2 = _CD * np.sqrt(2.0)

_CA_ROWS = [np.array([[v]] * 6 + [[1.0]] + [[0.0]], np.float32) for v in _CA]
_CB_ROWS = [np.array([[v]] * 6 + [[0.0]] + [[0.0]], np.float32) for v in _CB]
_CD_ROWS = [np.array([[v]] * 6 + [[0.0]] + [[0.0]], np.float32)
            for v in _CD_SQRT2]

_ALPHA = 1.0
for _ in range(80):
    _s = 1.0 / (1.0 + np.exp(-_ALPHA))
    _f = _ALPHA * _s - 1.0
    _fp = _s + _ALPHA * _s * (1.0 - _s)
    _ALPHA = _ALPHA - _f / _fp
_ALPHA = float(_ALPHA)

_U_LO = float(np.nextafter(np.float32(-1.0), np.float32(0.0)))
_U_SCALE = float(np.float32(1.0) - np.float32(_U_LO))

_KS2 = np.uint32(0x1BD11BDA ^ 0 ^ 1)


def _rotl(v, r):
    return (v << np.uint32(r)) | (v >> np.uint32(32 - r))


def _threefry_bits(cnt):
    x1 = cnt + np.uint32(1)
    x0 = x1
    x1 = _rotl(x1, 13) ^ x0
    for r in (15, 26, 6):
        x0 = x0 + x1
        x1 = _rotl(x1, r) ^ x0
    x0 = x0 + np.uint32(1)
    x1 = x1 + (_KS2 + np.uint32(1))
    for r in (17, 29, 16, 24):
        x0 = x0 + x1
        x1 = _rotl(x1, r) ^ x0
    x0 = x0 + _KS2
    x1 = x1 + np.uint32(2)
    for r in (13, 15, 26, 6):
        x0 = x0 + x1
        x1 = _rotl(x1, r) ^ x0
    x1 = x1 + np.uint32(4)
    for r in (17, 29, 16, 24):
        x0 = x0 + x1
        x1 = _rotl(x1, r) ^ x0
    x0 = x0 + np.uint32(1)
    x1 = x1 + (_KS2 + np.uint32(4))
    for r in (13, 15, 26, 6):
        x0 = x0 + x1
        x1 = _rotl(x1, r) ^ x0
    x0 = x0 + _KS2
    x1 = x1 + np.uint32(5)
    return x0 ^ x1


def _erfinv_poly(u):
    w = -jnp.log1p(-u * u)
    wl = w - np.float32(2.5)
    p1 = jnp.float32(2.81022636e-08)
    for c in (3.43273939e-07, -3.5233877e-06, -4.39150654e-06, 0.00021858087,
              -0.00125372503, -0.00417768164, 0.246640727, 1.50140941):
        p1 = p1 * wl + np.float32(c)
    wg = jnp.sqrt(w) - np.float32(3.0)
    p2 = jnp.float32(-0.000200214257)
    for c in (0.000100950558, 0.00134934322, -0.00367342844, 0.00573950773,
              -0.0076224613, 0.00943887047, 1.00167406, 2.83297682):
        p2 = p2 * wg + np.float32(c)
    return jnp.where(w < np.float32(5.0), p1, p2) * u


def _gen_z(rowoff, base, nsteps):
    shape = (nsteps, 8, _LC)
    cnt = (rowoff
           + jax.lax.broadcasted_iota(jnp.int32, shape, 2)
           + base)
    bits = _threefry_bits(pltpu.bitcast(cnt, jnp.uint32))
    fbits = (bits >> np.uint32(9)) | np.uint32(0x3F800000)
    frac = pltpu.bitcast(fbits, jnp.float32) - np.float32(1.0)
    u = frac * np.float32(_U_SCALE) + np.float32(_U_LO)
    return _erfinv_poly(u)


def _sample_body(x_ref, w1_ref, w2_ref, coef_ref, roff_ref, out_ref, *, hw):
    nch = hw // _LC
    g = pl.program_id(0)
    gbase = g * np.int32(2 * _T * _C * hw)

    xs = [x_ref[0, :, ch * _LC:(ch + 1) * _LC] for ch in range(nch)]
    zhold = [None] * nch
    for s in range(_T):
        w1s = w1_ref[0, s]
        w2s = w2_ref[s]
        cav = coef_ref[s, 0]
        cbv = coef_ref[s, 1]
        cdv = coef_ref[s, 2]
        for ch in range(nch):
            x = xs[ch]
            hpre = jnp.einsum("hk,kl->hl", w1s, x.astype(jnp.bfloat16),
                              preferred_element_type=jnp.float32)
            sg = np.float32(0.5) * jnp.tanh(np.float32(0.5) * hpre) \
                + np.float32(0.5)
            h = (hpre * sg).astype(jnp.bfloat16)
            psi = jnp.einsum("ch,hl->cl", w2s, h,
                             preferred_element_type=jnp.float32)
            pred = jax.lax.clamp(jnp.float32(-1.0),
                                 np.float32(_C0[s]) * x + psi,
                                 jnp.float32(1.0))
            xn = cav * pred
            if s < _T - 1:
                if s % 2 == 0:
                    nz = 2 if s < 6 else 1
                    zz = _gen_z(roff_ref[s // 2, 0:nz],
                                gbase + ch * _LC, nz)
                    z = zz[0]
                    if nz == 2:
                        zhold[ch] = zz[1]
                else:
                    z = zhold[ch]
                xn = xn + cbv * x + cdv * z
            xs[ch] = xn
    for ch in range(nch):
        out_ref[0, :, ch * _LC:(ch + 1) * _LC] = xs[ch]


def kernel(W1, b1, W2, b2, temb, cls_emb, noise, labels):
    B = noise.shape[0]
    H, W = noise.shape[2], noise.shape[3]
    hw = H * W
    B2 = B // 2

    noise6 = noise.astype(jnp.float32).reshape(B2, 6, hw)
    x2 = jnp.concatenate(
        [noise6, jnp.ones((B2, 1, hw), jnp.float32),
         jnp.zeros((B2, 1, hw), jnp.float32)], axis=1)

    cls = cls_emb[labels.astype(jnp.int32)]
    temb_o = temb[::-1]
    bias = (b1[None, None, :] + temb_o[None, :, :]
            + cls[:, None, :])
    biasA = bias[0::2]
    biasB = bias[1::2]
    w1t = jnp.transpose(W1)
    w1b = jnp.broadcast_to(w1t[None, None], (B2, _T, _HID, _C))
    zc3 = jnp.zeros((B2, _T, _HID, _C), jnp.float32)
    zc1 = jnp.zeros((B2, _T, _HID, 1), jnp.float32)
    rowsA = jnp.concatenate([w1b, zc3, biasA[..., None], zc1], axis=3)
    rowsB = jnp.concatenate([zc3, w1b, biasB[..., None], zc1], axis=3)
    alpha_row = jnp.broadcast_to(
        jnp.asarray([0, 0, 0, 0, 0, 0, _ALPHA, 0], jnp.float32)[None, None,
                                                                None, :],
        (B2, _T, 1, 8))
    w1tab = jnp.concatenate([rowsA, rowsB, alpha_row],
                            axis=2).astype(jnp.bfloat16)

    w2t = jnp.transpose(W2)
    z32 = jnp.zeros((_C, _HID), jnp.float32)
    b2c = b2[:, None]
    rA = jnp.concatenate([w2t, z32, b2c], axis=1)
    rB = jnp.concatenate([z32, w2t, b2c], axis=1)
    w2base = jnp.concatenate([rA, rB, jnp.zeros((2, 65), jnp.float32)],
                             axis=0)
    w2tab = (jnp.asarray(-_C1, jnp.float32)[:, None, None]
             * w2base[None]).astype(jnp.bfloat16)

    coef_tab = jnp.asarray(
        np.stack([np.stack([_CA_ROWS[s], _CB_ROWS[s], _CD_ROWS[s]], axis=0)
                  for s in range(_T)], axis=0))

    D = _T * _C * hw
    chan = np.array([0, hw, 2 * hw, D, D + hw, D + 2 * hw, 0, 0], np.int64)
    roff_tab = jnp.asarray(np.stack(
        [np.stack([chan + (s + p) * _C * hw for p in range(2)], axis=0)
         for s in (0, 2, 4, 6)], axis=0)[:, :, :, None].astype(np.int32))

    body = functools.partial(_sample_body, hw=hw)
    n_px = B * hw
    n_noise = int(np.sum(_CD != 0.0))
    out = pl.pallas_call(
        body,
        grid=(B2,),
        in_specs=[
            pl.BlockSpec((1, 8, hw), lambda gi: (gi, 0, 0)),
            pl.BlockSpec((1, _T, 2 * _HID + 1, 8), lambda gi: (gi, 0, 0, 0)),
            pl.BlockSpec((_T, 8, 2 * _HID + 1), lambda gi: (0, 0, 0)),
            pl.BlockSpec((_T, 3, 8, 1), lambda gi: (0, 0, 0, 0)),
            pl.BlockSpec((4, 2, 8, 1), lambda gi: (0, 0, 0, 0)),
        ],
        out_specs=pl.BlockSpec((1, 8, hw), lambda gi: (gi, 0, 0)),
        out_shape=jax.ShapeDtypeStruct((B2, 8, hw), jnp.float32),
        compiler_params=pltpu.CompilerParams(
            dimension_semantics=("parallel",)),
        cost_estimate=pl.CostEstimate(
            flops=int(_T * n_px * (2 * 8 * _HID + 2 * (_HID + 1) * _C
                                   + 16 * _C) + n_noise * n_px * _C * 45),
            transcendentals=int(_T * n_px * _HID + 2 * n_noise * n_px * _C),
            bytes_accessed=int(4 * (2 * B2 * 8 * hw
                                    + B2 * _T * (2 * _HID + 1) * 8
                                    + _T * 8 * (2 * _HID + 1))),
        ),
    )(x2, w1tab, w2tab, coef_tab, roff_tab)
    return out[:, 0:6, :].reshape(B, _C, H, W)


# speedup vs baseline: 2.1286x; 1.0600x over previous
import functools

import numpy as np
import jax
import jax.numpy as jnp
from jax.experimental import pallas as pl
from jax.experimental.pallas import tpu as pltpu

_C = 3
_HID = 32
_T = 8
_LC = 2048

_betas = np.linspace(1e-4, 2e-2, _T, dtype=np.float64)
_alphas = 1.0 - _betas
_abar = np.cumprod(_alphas)
_abar_prev = np.append(1.0, _abar[:-1])
_order = np.arange(_T - 1, -1, -1)

_C0 = np.sqrt(1.0 / _abar)[_order]
_C1 = np.sqrt(1.0 / _abar - 1.0)[_order]
_CA = (_betas * np.sqrt(_abar_prev) / (1.0 - _abar))[_order]
_CB = ((1.0 - _abar_prev) * np.sqrt(_alphas) / (1.0 - _abar))[_order]
_CD = np.sqrt(_betas * (1.0 - _abar_prev) / (1.0 - _abar))
_CD[0] = 0.0
_CD = _CD[_order]
_CD_SQRT2 = _CD * np.sqrt(2.0)

_CA_ROWS = [np.array([[v]] * 6 + [[1.0]] + [[0.0]], np.float32) for v in _CA]
_CB_ROWS = [np.array([[v]] * 6 + [[0.0]] + [[0.0]], np.float32) for v in _CB]
_CD_ROWS = [np.array([[v]] * 6 + [[0.0]] + [[0.0]], np.float32)
            for v in _CD_SQRT2]

_ALPHA = 1.0
for _ in range(80):
    _s = 1.0 / (1.0 + np.exp(-_ALPHA))
    _f = _ALPHA * _s - 1.0
    _fp = _s + _ALPHA * _s * (1.0 - _s)
    _ALPHA = _ALPHA - _f / _fp
_ALPHA = float(_ALPHA)

_U_LO = float(np.nextafter(np.float32(-1.0), np.float32(0.0)))
_U_SCALE = float(np.float32(1.0) - np.float32(_U_LO))

_KS2 = np.uint32(0x1BD11BDA ^ 0 ^ 1)


def _rotl(v, r):
    return (v << np.uint32(r)) | (v >> np.uint32(32 - r))


def _threefry_bits(cnt):
    x1 = cnt + np.uint32(1)
    x0 = x1
    x1 = _rotl(x1, 13) ^ x0
    for r in (15, 26, 6):
        x0 = x0 + x1
        x1 = _rotl(x1, r) ^ x0
    x0 = x0 + np.uint32(1)
    x1 = x1 + (_KS2 + np.uint32(1))
    for r in (17, 29, 16, 24):
        x0 = x0 + x1
        x1 = _rotl(x1, r) ^ x0
    x0 = x0 + _KS2
    x1 = x1 + np.uint32(2)
    for r in (13, 15, 26, 6):
        x0 = x0 + x1
        x1 = _rotl(x1, r) ^ x0
    x1 = x1 + np.uint32(4)
    for r in (17, 29, 16, 24):
        x0 = x0 + x1
        x1 = _rotl(x1, r) ^ x0
    x0 = x0 + np.uint32(1)
    x1 = x1 + (_KS2 + np.uint32(4))
    for r in (13, 15, 26, 6):
        x0 = x0 + x1
        x1 = _rotl(x1, r) ^ x0
    x0 = x0 + _KS2
    x1 = x1 + np.uint32(5)
    return x0 ^ x1


def _erfinv_poly(u):
    w = -jnp.log1p(-u * u)
    wl = w - np.float32(2.5)
    p1 = jnp.float32(2.81022636e-08)
    for c in (3.43273939e-07, -3.5233877e-06, -4.39150654e-06, 0.00021858087,
              -0.00125372503, -0.00417768164, 0.246640727, 1.50140941):
        p1 = p1 * wl + np.float32(c)
    wg = jnp.sqrt(w) - np.float32(3.0)
    p2 = jnp.float32(-0.000200214257)
    for c in (0.000100950558, 0.00134934322, -0.00367342844, 0.00573950773,
              -0.0076224613, 0.00943887047, 1.00167406, 2.83297682):
        p2 = p2 * wg + np.float32(c)
    return jnp.where(w < np.float32(5.0), p1, p2) * u


def _gen_z(rowoff, base, nsteps):
    shape = (nsteps, 8, _LC)
    cnt = (rowoff
           + jax.lax.broadcasted_iota(jnp.int32, shape, 2)
           + base)
    bits = _threefry_bits(pltpu.bitcast(cnt, jnp.uint32))
    fbits = (bits >> np.uint32(9)) | np.uint32(0x3F800000)
    frac = pltpu.bitcast(fbits, jnp.float32) - np.float32(1.0)
    u = frac * np.float32(_U_SCALE) + np.float32(_U_LO)
    return _erfinv_poly(u)


def _sample_body(x_ref, w1_ref, w2_ref, coef_ref, roff_ref, out_ref, *, hw):
    nch = hw // _LC
    g = pl.program_id(0)
    gbase = g * np.int32(2 * _T * _C * hw)

    xs = [x_ref[0, :, ch * _LC:(ch + 1) * _LC] for ch in range(nch)]
    zhold = [None] * nch
    for s in range(_T):
        w1s = w1_ref[0, s]
        w2s = w2_ref[s]
        cav = coef_ref[s, 0]
        cbv = coef_ref[s, 1]
        cdv = coef_ref[s, 2]
        cbiasv = coef_ref[s, 3]
        for ch in range(nch):
            x = xs[ch]
            hpre = jnp.einsum("hk,kl->hl", w1s, x.astype(jnp.bfloat16),
                              preferred_element_type=jnp.float32)
            t = np.float32(0.5) * hpre
            h = (t * jnp.tanh(t) + t).astype(jnp.bfloat16)
            psi = jnp.einsum("ch,hl->cl", w2s, h,
                             preferred_element_type=jnp.float32)
            pred = jax.lax.clamp(jnp.float32(-1.0),
                                 np.float32(_C0[s]) * x + (psi + cbiasv),
                                 jnp.float32(1.0))
            xn = cav * pred
            if s < _T - 1:
                if s % 2 == 0:
                    nz = 2 if s < 6 else 1
                    zz = _gen_z(roff_ref[s // 2, 0:nz],
                                gbase + ch * _LC, nz)
                    z = zz[0]
                    if nz == 2:
                        zhold[ch] = zz[1]
                else:
                    z = zhold[ch]
                xn = xn + cbv * x + cdv * z
            xs[ch] = xn
    for ch in range(nch):
        out_ref[0, :, ch * _LC:(ch + 1) * _LC] = xs[ch]


def kernel(W1, b1, W2, b2, temb, cls_emb, noise, labels):
    B = noise.shape[0]
    H, W = noise.shape[2], noise.shape[3]
    hw = H * W
    B2 = B // 2

    noise6 = noise.astype(jnp.float32).reshape(B2, 6, hw)
    x2 = jnp.concatenate(
        [noise6, jnp.ones((B2, 1, hw), jnp.float32),
         jnp.zeros((B2, 1, hw), jnp.float32)], axis=1)

    cls = cls_emb[labels.astype(jnp.int32)]
    temb_o = temb[::-1]
    bias = (b1[None, None, :] + temb_o[None, :, :]
            + cls[:, None, :])
    biasA = bias[0::2]
    biasB = bias[1::2]
    w1t = jnp.transpose(W1)
    w1b = jnp.broadcast_to(w1t[None, None], (B2, _T, _HID, _C))
    zc3 = jnp.zeros((B2, _T, _HID, _C), jnp.float32)
    zc1 = jnp.zeros((B2, _T, _HID, 1), jnp.float32)
    rowsA = jnp.concatenate([w1b, zc3, biasA[..., None], zc1], axis=3)
    rowsB = jnp.concatenate([zc3, w1b, biasB[..., None], zc1], axis=3)
    w1tab = jnp.concatenate([rowsA, rowsB],
                            axis=2).astype(jnp.bfloat16)

    w2t = jnp.transpose(W2)
    z32 = jnp.zeros((_C, _HID), jnp.float32)
    rA = jnp.concatenate([w2t, z32], axis=1)
    rB = jnp.concatenate([z32, w2t], axis=1)
    w2base = jnp.concatenate([rA, rB, jnp.zeros((2, 64), jnp.float32)],
                             axis=0)
    w2tab = (jnp.asarray(-_C1, jnp.float32)[:, None, None]
             * w2base[None]).astype(jnp.bfloat16)

    coef_np = jnp.asarray(
        np.stack([np.stack([_CA_ROWS[s], _CB_ROWS[s], _CD_ROWS[s]], axis=0)
                  for s in range(_T)], axis=0))
    cb2 = (jnp.asarray(-_C1, jnp.float32)[:, None] * b2[None, :])
    cbias = jnp.concatenate(
        [cb2, cb2, jnp.zeros((_T, 2), jnp.float32)], axis=1)
    coef_tab = jnp.concatenate(
        [coef_np, cbias[:, None, :, None]], axis=1)

    D = _T * _C * hw
    chan = np.array([0, hw, 2 * hw, D, D + hw, D + 2 * hw, 0, 0], np.int64)
    roff_tab = jnp.asarray(np.stack(
        [np.stack([chan + (s + p) * _C * hw for p in range(2)], axis=0)
         for s in (0, 2, 4, 6)], axis=0)[:, :, :, None].astype(np.int32))

    body = functools.partial(_sample_body, hw=hw)
    n_px = B * hw
    n_noise = int(np.sum(_CD != 0.0))
    out = pl.pallas_call(
        body,
        grid=(B2,),
        in_specs=[
            pl.BlockSpec((1, 8, hw), lambda gi: (gi, 0, 0)),
            pl.BlockSpec((1, _T, 2 * _HID, 8), lambda gi: (gi, 0, 0, 0)),
            pl.BlockSpec((_T, 8, 2 * _HID), lambda gi: (0, 0, 0)),
            pl.BlockSpec((_T, 4, 8, 1), lambda gi: (0, 0, 0, 0)),
            pl.BlockSpec((4, 2, 8, 1), lambda gi: (0, 0, 0, 0)),
        ],
        out_specs=pl.BlockSpec((1, 8, hw), lambda gi: (gi, 0, 0)),
        out_shape=jax.ShapeDtypeStruct((B2, 8, hw), jnp.float32),
        compiler_params=pltpu.CompilerParams(
            dimension_semantics=("parallel",)),
        cost_estimate=pl.CostEstimate(
            flops=int(_T * n_px * (2 * 8 * _HID + 2 * (_HID + 1) * _C
                                   + 16 * _C) + n_noise * n_px * _C * 45),
            transcendentals=int(_T * n_px * _HID + 2 * n_noise * n_px * _C),
            bytes_accessed=int(4 * (2 * B2 * 8 * hw
                                    + B2 * _T * (2 * _HID + 1) * 8
                                    + _T * 8 * (2 * _HID + 1))),
        ),
    )(x2, w1tab, w2tab, coef_tab, roff_tab)
    return out[:, 0:6, :].reshape(B, _C, H, W)


# speedup vs baseline: 2.1360x; 1.0035x over previous
import functools

import numpy as np
import jax
import jax.numpy as jnp
from jax.experimental import pallas as pl
from jax.experimental.pallas import tpu as pltpu

_C = 3
_HID = 32
_T = 8
_LC = 2048

_betas = np.linspace(1e-4, 2e-2, _T, dtype=np.float64)
_alphas = 1.0 - _betas
_abar = np.cumprod(_alphas)
_abar_prev = np.append(1.0, _abar[:-1])
_order = np.arange(_T - 1, -1, -1)

_C0 = np.sqrt(1.0 / _abar)[_order]
_C1 = np.sqrt(1.0 / _abar - 1.0)[_order]
_CA = (_betas * np.sqrt(_abar_prev) / (1.0 - _abar))[_order]
_CB = ((1.0 - _abar_prev) * np.sqrt(_alphas) / (1.0 - _abar))[_order]
_CD = np.sqrt(_betas * (1.0 - _abar_prev) / (1.0 - _abar))
_CD[0] = 0.0
_CD = _CD[_order]
_CD_SQRT2 = _CD * np.sqrt(2.0)

_CA_ROWS = [np.array([[v]] * 6 + [[1.0]] + [[0.0]], np.float32) for v in _CA]
_CB_ROWS = [np.array([[v]] * 6 + [[0.0]] + [[0.0]], np.float32) for v in _CB]
_CD_ROWS = [np.array([[v]] * 6 + [[0.0]] + [[0.0]], np.float32)
            for v in _CD_SQRT2]

_U_LO = float(np.nextafter(np.float32(-1.0), np.float32(0.0)))
_U_SCALE = float(np.float32(1.0) - np.float32(_U_LO))

_KS2 = np.uint32(0x1BD11BDA ^ 0 ^ 1)


def _rotl(v, r):
    return (v << np.uint32(r)) | (v >> np.uint32(32 - r))


def _threefry_bits(cnt):
    x1 = cnt + np.uint32(1)
    x0 = x1
    x1 = _rotl(x1, 13) ^ x0
    for r in (15, 26, 6):
        x0 = x0 + x1
        x1 = _rotl(x1, r) ^ x0
    x0 = x0 + np.uint32(1)
    x1 = x1 + (_KS2 + np.uint32(1))
    for r in (17, 29, 16, 24):
        x0 = x0 + x1
        x1 = _rotl(x1, r) ^ x0
    x0 = x0 + _KS2
    x1 = x1 + np.uint32(2)
    for r in (13, 15, 26, 6):
        x0 = x0 + x1
        x1 = _rotl(x1, r) ^ x0
    x1 = x1 + np.uint32(4)
    for r in (17, 29, 16, 24):
        x0 = x0 + x1
        x1 = _rotl(x1, r) ^ x0
    x0 = x0 + np.uint32(1)
    x1 = x1 + (_KS2 + np.uint32(4))
    for r in (13, 15, 26, 6):
        x0 = x0 + x1
        x1 = _rotl(x1, r) ^ x0
    x0 = x0 + _KS2
    x1 = x1 + np.uint32(5)
    return x0 ^ x1


def _erfinv_poly(u):
    w = -jnp.log1p(-u * u)
    wl = w - np.float32(2.5)
    p1 = jnp.float32(2.81022636e-08)
    for c in (3.43273939e-07, -3.5233877e-06, -4.39150654e-06, 0.00021858087,
              -0.00125372503, -0.00417768164, 0.246640727, 1.50140941):
        p1 = p1 * wl + np.float32(c)
    wg = jnp.sqrt(w) - np.float32(3.0)
    p2 = jnp.float32(-0.000200214257)
    for c in (0.000100950558, 0.00134934322, -0.00367342844, 0.00573950773,
              -0.0076224613, 0.00943887047, 1.00167406, 2.83297682):
        p2 = p2 * wg + np.float32(c)
    return jnp.where(w < np.float32(5.0), p1, p2) * u


def _gen_z(rowoff, base, nsteps):
    shape = (nsteps, 8, _LC)
    cnt = (rowoff
           + jax.lax.broadcasted_iota(jnp.int32, shape, 2)
           + base)
    bits = _threefry_bits(pltpu.bitcast(cnt, jnp.uint32))
    fbits = (bits >> np.uint32(9)) | np.uint32(0x3F800000)
    frac = pltpu.bitcast(fbits, jnp.float32) - np.float32(1.0)
    u = frac * np.float32(_U_SCALE) + np.float32(_U_LO)
    return _erfinv_poly(u)


def _sample_body(x_ref, w1_ref, w2_ref, coef_ref, roff_ref, out_ref, *, hw):
    nch = hw // _LC
    g = pl.program_id(0)
    gbase = g * np.int32(2 * _T * _C * hw)

    xs = [x_ref[0, :, ch * _LC:(ch + 1) * _LC] for ch in range(nch)]
    zhold = [None] * nch
    for s in range(_T):
        w1s = w1_ref[0, s]
        w2s = w2_ref[s]
        cav = coef_ref[s, 0]
        cbv = coef_ref[s, 1]
        cdv = coef_ref[s, 2]
        cbiasv = coef_ref[s, 3]
        for ch in range(nch):
            x = xs[ch]
            hpre = jnp.einsum("hk,kl->hl", w1s, x.astype(jnp.bfloat16),
                              preferred_element_type=jnp.float32)
            t = np.float32(0.5) * hpre
            h = (t * jnp.tanh(t) + t).astype(jnp.bfloat16)
            psi = jnp.einsum("ch,hl->cl", w2s, h,
                             preferred_element_type=jnp.float32)
            pred = jax.lax.clamp(jnp.float32(-1.0),
                                 np.float32(_C0[s]) * x + (psi + cbiasv),
                                 jnp.float32(1.0))
            xn = cav * pred
            if s < _T - 1:
                if s % 2 == 0:
                    nz = 2 if s < 6 else 1
                    zz = _gen_z(roff_ref[s // 2, 0:nz],
                                gbase + ch * _LC, nz)
                    z = zz[0]
                    if nz == 2:
                        zhold[ch] = zz[1]
                else:
                    z = zhold[ch]
                xn = xn + cbv * x + cdv * z
            xs[ch] = xn
    for ch in range(nch):
        out_ref[0, :, ch * _LC:(ch + 1) * _LC] = xs[ch]


def kernel(W1, b1, W2, b2, temb, cls_emb, noise, labels):
    B = noise.shape[0]
    H, W = noise.shape[2], noise.shape[3]
    hw = H * W
    B2 = B // 2

    noise6 = noise.astype(jnp.float32).reshape(B2, 6, hw)
    x2 = jnp.concatenate(
        [noise6, jnp.ones((B2, 1, hw), jnp.float32),
         jnp.zeros((B2, 1, hw), jnp.float32)], axis=1)

    cls = cls_emb[labels.astype(jnp.int32)]
    temb_o = temb[::-1]
    bias = (b1[None, None, :] + temb_o[None, :, :]
            + cls[:, None, :])
    biasA = bias[0::2]
    biasB = bias[1::2]
    w1t = jnp.transpose(W1)
    w1b = jnp.broadcast_to(w1t[None, None], (B2, _T, _HID, _C))
    zc3 = jnp.zeros((B2, _T, _HID, _C), jnp.float32)
    zc1 = jnp.zeros((B2, _T, _HID, 1), jnp.float32)
    rowsA = jnp.concatenate([w1b, zc3, biasA[..., None], zc1], axis=3)
    rowsB = jnp.concatenate([zc3, w1b, biasB[..., None], zc1], axis=3)
    w1tab = jnp.concatenate([rowsA, rowsB],
                            axis=2).astype(jnp.bfloat16)

    w2t = jnp.transpose(W2)
    z32 = jnp.zeros((_C, _HID), jnp.float32)
    rA = jnp.concatenate([w2t, z32], axis=1)
    rB = jnp.concatenate([z32, w2t], axis=1)
    w2base = jnp.concatenate([rA, rB, jnp.zeros((2, 64), jnp.float32)],
                             axis=0)
    w2tab = (jnp.asarray(-_C1, jnp.float32)[:, None, None]
             * w2base[None]).astype(jnp.bfloat16)

    coef_np = jnp.asarray(
        np.stack([np.stack([_CA_ROWS[s], _CB_ROWS[s], _CD_ROWS[s]], axis=0)
                  for s in range(_T)], axis=0))
    cb2 = (jnp.asarray(-_C1, jnp.float32)[:, None] * b2[None, :])
    cbias = jnp.concatenate(
        [cb2, cb2, jnp.zeros((_T, 2), jnp.float32)], axis=1)
    coef_tab = jnp.concatenate(
        [coef_np, cbias[:, None, :, None]], axis=1)

    D = _T * _C * hw
    chan = np.array([0, hw, 2 * hw, D, D + hw, D + 2 * hw, 0, 0], np.int64)
    roff_tab = jnp.asarray(np.stack(
        [np.stack([chan + (s + p) * _C * hw for p in range(2)], axis=0)
         for s in (0, 2, 4, 6)], axis=0)[:, :, :, None].astype(np.int32))

    body = functools.partial(_sample_body, hw=hw)
    n_px = B * hw
    n_noise = int(np.sum(_CD != 0.0))
    out = pl.pallas_call(
        body,
        grid=(B2,),
        in_specs=[
            pl.BlockSpec((1, 8, hw), lambda gi: (gi, 0, 0)),
            pl.BlockSpec((1, _T, 2 * _HID, 8), lambda gi: (gi, 0, 0, 0)),
            pl.BlockSpec((_T, 8, 2 * _HID), lambda gi: (0, 0, 0)),
            pl.BlockSpec((_T, 4, 8, 1), lambda gi: (0, 0, 0, 0)),
            pl.BlockSpec((4, 2, 8, 1), lambda gi: (0, 0, 0, 0)),
        ],
        out_specs=pl.BlockSpec((1, 8, hw), lambda gi: (gi, 0, 0)),
        out_shape=jax.ShapeDtypeStruct((B2, 8, hw), jnp.float32),
        compiler_params=pltpu.CompilerParams(
            dimension_semantics=("parallel",)),
        cost_estimate=pl.CostEstimate(
            flops=int(_T * n_px * (2 * 8 * _HID + 2 * (_HID + 1) * _C
                                   + 16 * _C) + n_noise * n_px * _C * 45),
            transcendentals=int(_T * n_px * _HID + 2 * n_noise * n_px * _C),
            bytes_accessed=int(4 * (2 * B2 * 8 * hw
                                    + B2 * _T * (2 * _HID + 1) * 8
                                    + _T * 8 * (2 * _HID + 1))),
        ),
    )(x2, w1tab, w2tab, coef_tab, roff_tab)
    return out[:, 0:6, :].reshape(B, _C, H, W)


# speedup vs baseline: 2.2978x; 1.0757x over previous
import functools

import numpy as np
import jax
import jax.numpy as jnp
from jax.experimental import pallas as pl
from jax.experimental.pallas import tpu as pltpu

_C = 3
_HID = 32
_T = 8
_LC = 2048

_betas = np.linspace(1e-4, 2e-2, _T, dtype=np.float64)
_alphas = 1.0 - _betas
_abar = np.cumprod(_alphas)
_abar_prev = np.append(1.0, _abar[:-1])
_order = np.arange(_T - 1, -1, -1)

_C0 = np.sqrt(1.0 / _abar)[_order]
_C1 = np.sqrt(1.0 / _abar - 1.0)[_order]
_CA = (_betas * np.sqrt(_abar_prev) / (1.0 - _abar))[_order]
_CB = ((1.0 - _abar_prev) * np.sqrt(_alphas) / (1.0 - _abar))[_order]
_CD = np.sqrt(_betas * (1.0 - _abar_prev) / (1.0 - _abar))
_CD[0] = 0.0
_CD = _CD[_order]
_CD_SQRT2 = _CD * np.sqrt(2.0)

_CA_ROWS = [np.array([[v]] * 6 + [[1.0]] + [[0.0]], np.float32) for v in _CA]
_CB_ROWS = [np.array([[v]] * 6 + [[0.0]] + [[0.0]], np.float32) for v in _CB]
_CD_ROWS = [np.array([[v]] * 6 + [[0.0]] + [[0.0]], np.float32)
            for v in _CD_SQRT2]

_U_LO = float(np.nextafter(np.float32(-1.0), np.float32(0.0)))
_U_SCALE = float(np.float32(1.0) - np.float32(_U_LO))

_KS2 = np.uint32(0x1BD11BDA ^ 0 ^ 1)


def _rotl(v, r):
    return (v << np.uint32(r)) | (v >> np.uint32(32 - r))


def _threefry_bits(cnt):
    x1 = cnt + np.uint32(1)
    x0 = x1
    x1 = _rotl(x1, 13) ^ x0
    for r in (15, 26, 6):
        x0 = x0 + x1
        x1 = _rotl(x1, r) ^ x0
    x0 = x0 + np.uint32(1)
    x1 = x1 + (_KS2 + np.uint32(1))
    for r in (17, 29, 16, 24):
        x0 = x0 + x1
        x1 = _rotl(x1, r) ^ x0
    x0 = x0 + _KS2
    x1 = x1 + np.uint32(2)
    for r in (13, 15, 26, 6):
        x0 = x0 + x1
        x1 = _rotl(x1, r) ^ x0
    x1 = x1 + np.uint32(4)
    for r in (17, 29, 16, 24):
        x0 = x0 + x1
        x1 = _rotl(x1, r) ^ x0
    x0 = x0 + np.uint32(1)
    x1 = x1 + (_KS2 + np.uint32(4))
    for r in (13, 15, 26, 6):
        x0 = x0 + x1
        x1 = _rotl(x1, r) ^ x0
    x0 = x0 + _KS2
    x1 = x1 + np.uint32(5)
    return x0 ^ x1


def _erfinv_poly(u):
    w = -jnp.log1p(-u * u)
    wl = w - np.float32(2.5)
    p1 = jnp.float32(0.0001901627256302163)
    for c in (-0.0012699998915195465, -0.004119594115763903,
              0.24665617942810059, 1.5013922452926636):
        p1 = p1 * wl + np.float32(c)
    wg = jnp.sqrt(w) - np.float32(3.0)
    p2 = jnp.float32(-0.006716672331094742)
    for c in (0.012801758013665676, 1.000846028327942, 2.8327476978302):
        p2 = p2 * wg + np.float32(c)
    return jnp.where(w < np.float32(5.0), p1, p2) * u


def _gen_z(rowoff, base, nsteps):
    shape = (nsteps, 8, _LC)
    cnt = (rowoff
           + jax.lax.broadcasted_iota(jnp.int32, shape, 2)
           + base)
    bits = _threefry_bits(pltpu.bitcast(cnt, jnp.uint32))
    fbits = (bits >> np.uint32(9)) | np.uint32(0x3F800000)
    frac = pltpu.bitcast(fbits, jnp.float32) - np.float32(1.0)
    u = frac * np.float32(_U_SCALE) + np.float32(_U_LO)
    return _erfinv_poly(u)


def _sample_body(x_ref, w1_ref, w2_ref, coef_ref, roff_ref, out_ref, *, hw):
    nch = hw // _LC
    g = pl.program_id(0)
    gbase = g * np.int32(2 * _T * _C * hw)

    xs = [x_ref[0, :, ch * _LC:(ch + 1) * _LC] for ch in range(nch)]
    zhold = [None] * nch
    for s in range(_T):
        w1s = w1_ref[0, s]
        w2s = w2_ref[s]
        cav = coef_ref[s, 0]
        cbv = coef_ref[s, 1]
        cdv = coef_ref[s, 2]
        cbiasv = coef_ref[s, 3]
        for ch in range(nch):
            x = xs[ch]
            hpre = jnp.einsum("hk,kl->hl", w1s, x.astype(jnp.bfloat16),
                              preferred_element_type=jnp.float32)
            t = np.float32(0.5) * hpre
            h = (t * jnp.tanh(t) + t).astype(jnp.bfloat16)
            psi = jnp.einsum("ch,hl->cl", w2s, h,
                             preferred_element_type=jnp.float32)
            pred = jax.lax.clamp(jnp.float32(-1.0),
                                 np.float32(_C0[s]) * x + (psi + cbiasv),
                                 jnp.float32(1.0))
            xn = cav * pred
            if s < _T - 1:
                if s % 2 == 0:
                    nz = 2 if s < 6 else 1
                    zz = _gen_z(roff_ref[s // 2, 0:nz],
                                gbase + ch * _LC, nz)
                    z = zz[0]
                    if nz == 2:
                        zhold[ch] = zz[1]
                else:
                    z = zhold[ch]
                xn = xn + cbv * x + cdv * z
            xs[ch] = xn
    for ch in range(nch):
        out_ref[0, :, ch * _LC:(ch + 1) * _LC] = xs[ch]


def kernel(W1, b1, W2, b2, temb, cls_emb, noise, labels):
    B = noise.shape[0]
    H, W = noise.shape[2], noise.shape[3]
    hw = H * W
    B2 = B // 2

    noise6 = noise.astype(jnp.float32).reshape(B2, 6, hw)
    x2 = jnp.concatenate(
        [noise6, jnp.ones((B2, 1, hw), jnp.float32),
         jnp.zeros((B2, 1, hw), jnp.float32)], axis=1)

    cls = cls_emb[labels.astype(jnp.int32)]
    temb_o = temb[::-1]
    bias = (b1[None, None, :] + temb_o[None, :, :]
            + cls[:, None, :])
    biasA = bias[0::2]
    biasB = bias[1::2]
    w1t = jnp.transpose(W1)
    w1b = jnp.broadcast_to(w1t[None, None], (B2, _T, _HID, _C))
    zc3 = jnp.zeros((B2, _T, _HID, _C), jnp.float32)
    zc1 = jnp.zeros((B2, _T, _HID, 1), jnp.float32)
    rowsA = jnp.concatenate([w1b, zc3, biasA[..., None], zc1], axis=3)
    rowsB = jnp.concatenate([zc3, w1b, biasB[..., None], zc1], axis=3)
    w1tab = jnp.concatenate([rowsA, rowsB],
                            axis=2).astype(jnp.bfloat16)

    w2t = jnp.transpose(W2)
    z32 = jnp.zeros((_C, _HID), jnp.float32)
    rA = jnp.concatenate([w2t, z32], axis=1)
    rB = jnp.concatenate([z32, w2t], axis=1)
    w2base = jnp.concatenate([rA, rB, jnp.zeros((2, 64), jnp.float32)],
                             axis=0)
    w2tab = (jnp.asarray(-_C1, jnp.float32)[:, None, None]
             * w2base[None]).astype(jnp.bfloat16)

    coef_np = jnp.asarray(
        np.stack([np.stack([_CA_ROWS[s], _CB_ROWS[s], _CD_ROWS[s]], axis=0)
                  for s in range(_T)], axis=0))
    cb2 = (jnp.asarray(-_C1, jnp.float32)[:, None] * b2[None, :])
    cbias = jnp.concatenate(
        [cb2, cb2, jnp.zeros((_T, 2), jnp.float32)], axis=1)
    coef_tab = jnp.concatenate(
        [coef_np, cbias[:, None, :, None]], axis=1)

    D = _T * _C * hw
    chan = np.array([0, hw, 2 * hw, D, D + hw, D + 2 * hw, 0, 0], np.int64)
    roff_tab = jnp.asarray(np.stack(
        [np.stack([chan + (s + p) * _C * hw for p in range(2)], axis=0)
         for s in (0, 2, 4, 6)], axis=0)[:, :, :, None].astype(np.int32))

    body = functools.partial(_sample_body, hw=hw)
    n_px = B * hw
    n_noise = int(np.sum(_CD != 0.0))
    out = pl.pallas_call(
        body,
        grid=(B2,),
        in_specs=[
            pl.BlockSpec((1, 8, hw), lambda gi: (gi, 0, 0)),
            pl.BlockSpec((1, _T, 2 * _HID, 8), lambda gi: (gi, 0, 0, 0)),
            pl.BlockSpec((_T, 8, 2 * _HID), lambda gi: (0, 0, 0)),
            pl.BlockSpec((_T, 4, 8, 1), lambda gi: (0, 0, 0, 0)),
            pl.BlockSpec((4, 2, 8, 1), lambda gi: (0, 0, 0, 0)),
        ],
        out_specs=pl.BlockSpec((1, 8, hw), lambda gi: (gi, 0, 0)),
        out_shape=jax.ShapeDtypeStruct((B2, 8, hw), jnp.float32),
        compiler_params=pltpu.CompilerParams(
            dimension_semantics=("parallel",)),
        cost_estimate=pl.CostEstimate(
            flops=int(_T * n_px * (2 * 8 * _HID + 2 * (_HID + 1) * _C
                                   + 16 * _C) + n_noise * n_px * _C * 45),
            transcendentals=int(_T * n_px * _HID + 2 * n_noise * n_px * _C),
            bytes_accessed=int(4 * (2 * B2 * 8 * hw
                                    + B2 * _T * (2 * _HID + 1) * 8
                                    + _T * 8 * (2 * _HID + 1))),
        ),
    )(x2, w1tab, w2tab, coef_tab, roff_tab)
    return out[:, 0:6, :].reshape(B, _C, H, W)
